# baseline (speedup 1.0000x reference)
import jax
import jax.numpy as jnp
from jax import lax
from jax.experimental import pallas as pl
from jax.experimental.pallas import tpu as pltpu
from jax.experimental.pallas import tpu_sc as plsc

B = 16
N = 4096
D = 64
TEMPERATURE = 0.05

K_SC = 4
B_TC = B - K_SC

NW = 32
W_PER_ROW = NW // K_SC
POS_PER_W = (K_SC * N) // NW
CH = 256
NCH = POS_PER_W // CH
GROUPS = CH // 16

TCCH = 4096


def _softplus_neg_abs(absl):
    u = jnp.exp(-absl)
    z = u / (u + 2.0)
    z2 = z * z
    p = 1.0 / 7.0 + z2 * (1.0 / 9.0)
    p = 1.0 / 5.0 + z2 * p
    p = 1.0 / 3.0 + z2 * p
    return 2.0 * z * (1.0 + z2 * p)


def _sc_body(len_hbm, o_hbm, s_hbm, w_hbm, r_hbm, part_out,
             o_buf0, o_buf1, s_buf0, s_buf1, w_buf0, w_buf1, r_buf0, r_buf1,
             len_buf, acc_buf, sem0, sem1):
    cid = lax.axis_index("c")
    sid = lax.axis_index("s")
    wid = sid * 2 + cid
    bb = B_TC + wid // W_PER_ROW
    n0 = (wid % W_PER_ROW) * POS_PER_W

    pltpu.sync_copy(len_hbm, len_buf)
    lane = lax.broadcasted_iota(jnp.int32, (16,), 0)
    l_vec = jnp.take_along_axis(len_buf[...], jnp.full((16,), bb, jnp.int32), axis=0)

    obufs = (o_buf0, o_buf1)
    sbufs = (s_buf0, s_buf1)
    wbufs = (w_buf0, w_buf1)
    rbufs = (r_buf0, r_buf1)
    sems = (sem0, sem1)

    def copies(c, ph):
        nst = n0 + c * CH
        return (
            pltpu.make_async_copy(o_hbm.at[bb, :, pl.ds(nst, CH)], obufs[ph], sems[ph]),
            pltpu.make_async_copy(s_hbm.at[bb, :, pl.ds(nst, CH)], sbufs[ph], sems[ph]),
            pltpu.make_async_copy(w_hbm.at[bb, pl.ds(nst, CH)], wbufs[ph], sems[ph]),
            pltpu.make_async_copy(r_hbm.at[bb, pl.ds(nst, CH)], rbufs[ph], sems[ph]),
        )

    def start(c, ph):
        for cp in copies(c, ph):
            cp.start()

    def wait(c, ph):
        for cp in copies(c, ph):
            cp.wait()

    start(0, 0)
    start(1, 1)

    def compute_chunk(c, ph, wl_acc, w_acc):
        ob = obufs[ph]
        sb = sbufs[ph]

        def group_body(g, inner):
            wl_a, wl_c, w_a, w_c = inner
            off = g * 16

            def d_body(j, accs):
                d0 = j * 8
                return tuple(
                    accs[k] + ob[d0 + k, pl.ds(off, 16)] * sb[d0 + k, pl.ds(off, 16)]
                    for k in range(8)
                )

            zero16 = jnp.zeros((16,), jnp.float32)
            accs = lax.fori_loop(0, D // 8, d_body, (zero16,) * 8)
            acc = ((accs[0] + accs[1]) + (accs[2] + accs[3])) + (
                (accs[4] + accs[5]) + (accs[6] + accs[7])
            )
            logits = acc * (1.0 / TEMPERATURE)
            t = rbufs[ph][pl.ds(off, 16)]
            w_raw = wbufs[ph][pl.ds(off, 16)]
            n_vec = n0 + c * CH + off + lane
            w = jnp.where(n_vec < l_vec, w_raw, 0.0)
            bce = jnp.maximum(logits, 0.0) - logits * t + _softplus_neg_abs(jnp.abs(logits))

            y1 = bce * w - wl_c
            t1 = wl_a + y1
            wl_c_new = (t1 - wl_a) - y1
            y2 = w - w_c
            t2 = w_a + y2
            w_c_new = (t2 - w_a) - y2
            return t1, wl_c_new, t2, w_c_new

        return plsc.parallel_loop(
            0, GROUPS, carry=(wl_acc[0], wl_acc[1], w_acc[0], w_acc[1])
        )(group_body)

    def pair_body(i, carry):
        wl, wlc, w, wc = carry
        for ph in range(2):
            c = 2 * i + ph
            wait(c, ph)
            wl, wlc, w, wc = compute_chunk(c, ph, (wl, wlc), (w, wc))

            @pl.when(c + 2 < NCH)
            def _():
                start(c + 2, ph)

        return wl, wlc, w, wc

    zero = jnp.zeros((16,), jnp.float32)
    wl, _, w, _ = lax.fori_loop(0, NCH // 2, pair_body, (zero, zero, zero, zero))

    acc_buf[pl.ds(0, 16)] = wl
    acc_buf[pl.ds(16, 16)] = w
    pltpu.sync_copy(acc_buf, part_out.at[pl.ds(wid * 32, 32)])


def _tc_body(len_ref, o_ref, s_ref, w_ref, r_ref, wl_out, w_out):
    b = pl.program_id(0)
    j = pl.program_id(1)

    o = o_ref[0]
    s = s_ref[0]
    logits = jnp.sum(o * s, axis=0) * (1.0 / TEMPERATURE)

    n_idx = j * TCCH + lax.broadcasted_iota(jnp.int32, (TCCH,), 0)
    valid = (n_idx < len_ref[b]).astype(jnp.float32)
    w = w_ref[b] * valid
    t = r_ref[b]
    bce = jnp.maximum(logits, 0.0) - logits * t + jnp.log1p(jnp.exp(-jnp.abs(logits)))

    wl_out[0, 0] = bce * w
    w_out[0, 0] = w


@jax.jit
def _run(lengths, o_t, s_t, w2, r2):
    mesh = plsc.VectorSubcoreMesh(core_axis_name="c", subcore_axis_name="s")
    sc = pl.kernel(
        _sc_body,
        out_type=jax.ShapeDtypeStruct((NW * 32,), jnp.float32),
        mesh=mesh,
        scratch_types=[
            pltpu.VMEM((D, CH), jnp.float32),
            pltpu.VMEM((D, CH), jnp.float32),
            pltpu.VMEM((D, CH), jnp.float32),
            pltpu.VMEM((D, CH), jnp.float32),
            pltpu.VMEM((CH,), jnp.float32),
            pltpu.VMEM((CH,), jnp.float32),
            pltpu.VMEM((CH,), jnp.float32),
            pltpu.VMEM((CH,), jnp.float32),
            pltpu.VMEM((16,), jnp.int32),
            pltpu.VMEM((32,), jnp.float32),
            pltpu.SemaphoreType.DMA,
            pltpu.SemaphoreType.DMA,
        ],
        compiler_params=pltpu.CompilerParams(needs_layout_passes=False),
    )
    sc_parts = sc(lengths, o_t, s_t, w2, r2).reshape(NW, 2, 16)

    wl_tc, w_tc = pl.pallas_call(
        _tc_body,
        grid=(B_TC, N // TCCH),
        in_specs=[
            pl.BlockSpec(memory_space=pltpu.SMEM),
            pl.BlockSpec((1, D, TCCH), lambda b, j: (b, 0, j)),
            pl.BlockSpec((1, D, TCCH), lambda b, j: (b, 0, j)),
            pl.BlockSpec((B, N), lambda b, j: (0, 0)),
            pl.BlockSpec((B, N), lambda b, j: (0, 0)),
        ],
        out_specs=[
            pl.BlockSpec((1, 1, TCCH), lambda b, j: (b, 0, j)),
            pl.BlockSpec((1, 1, TCCH), lambda b, j: (b, 0, j)),
        ],
        out_shape=[
            jax.ShapeDtypeStruct((B_TC, 1, N), jnp.float32),
            jax.ShapeDtypeStruct((B_TC, 1, N), jnp.float32),
        ],
    )(lengths, o_t, s_t, w2, r2)

    num = jnp.sum(sc_parts[:, 0, :]) + jnp.sum(wl_tc)
    den = jnp.sum(sc_parts[:, 1, :]) + jnp.sum(w_tc)
    return num / den


def kernel(lengths, output_embeddings, supervision_ids, supervision_embeddings, supervision_weights, supervision_ratings):
    del supervision_ids
    o_t = output_embeddings.transpose(0, 2, 1)
    s_t = supervision_embeddings.transpose(0, 2, 1)
    return _run(lengths, o_t, s_t, supervision_weights, supervision_ratings)


# speedup vs baseline: 1.0890x; 1.0890x over previous
import jax
import jax.numpy as jnp
from jax import lax
from jax.experimental import pallas as pl
from jax.experimental.pallas import tpu as pltpu
from jax.experimental.pallas import tpu_sc as plsc

B = 16
N = 4096
D = 64
TEMPERATURE = 0.05

K_SC = 4
B_TC = B - K_SC

NW = 32
W_PER_ROW = NW // K_SC
POS_PER_W = (K_SC * N) // NW
CH = 256
NCH = POS_PER_W // CH
GROUPS = CH // 16

TCCH = 4096


def _softplus_neg_abs(absl):
    u = jnp.exp(-absl)
    z = u / (u + 2.0)
    z2 = z * z
    p = 1.0 / 7.0 + z2 * (1.0 / 9.0)
    p = 1.0 / 5.0 + z2 * p
    p = 1.0 / 3.0 + z2 * p
    return 2.0 * z * (1.0 + z2 * p)


def _sc_body(len_hbm, o_hbm, s_hbm, w_hbm, r_hbm, part_out,
             o_buf0, o_buf1, s_buf0, s_buf1, w_buf0, w_buf1, r_buf0, r_buf1,
             len_buf, acc_buf, sem0, sem1):
    cid = lax.axis_index("c")
    sid = lax.axis_index("s")
    wid = sid * 2 + cid
    bb = B_TC + wid // W_PER_ROW
    n0 = (wid % W_PER_ROW) * POS_PER_W

    pltpu.sync_copy(len_hbm, len_buf)
    lane = lax.broadcasted_iota(jnp.int32, (16,), 0)
    l_vec = jnp.take_along_axis(len_buf[...], jnp.full((16,), bb, jnp.int32), axis=0)

    obufs = (o_buf0, o_buf1)
    sbufs = (s_buf0, s_buf1)
    wbufs = (w_buf0, w_buf1)
    rbufs = (r_buf0, r_buf1)
    sems = (sem0, sem1)

    def copies(c, ph):
        nst = n0 + c * CH
        return (
            pltpu.make_async_copy(o_hbm.at[bb, :, pl.ds(nst, CH)], obufs[ph], sems[ph]),
            pltpu.make_async_copy(s_hbm.at[bb, :, pl.ds(nst, CH)], sbufs[ph], sems[ph]),
            pltpu.make_async_copy(w_hbm.at[bb, pl.ds(nst, CH)], wbufs[ph], sems[ph]),
            pltpu.make_async_copy(r_hbm.at[bb, pl.ds(nst, CH)], rbufs[ph], sems[ph]),
        )

    def start(c, ph):
        for cp in copies(c, ph):
            cp.start()

    def wait(c, ph):
        for cp in copies(c, ph):
            cp.wait()

    start(0, 0)
    start(1, 1)

    def compute_chunk(c, ph, wl_acc, w_acc):
        ob = obufs[ph]
        sb = sbufs[ph]

        def group_body(g, inner):
            wl_a, wl_c, w_a, w_c = inner
            off = g * 16

            def d_body(j, accs):
                d0 = j * 8
                return tuple(
                    accs[k] + ob[d0 + k, pl.ds(off, 16)] * sb[d0 + k, pl.ds(off, 16)]
                    for k in range(8)
                )

            zero16 = jnp.zeros((16,), jnp.float32)
            accs = lax.fori_loop(0, D // 8, d_body, (zero16,) * 8)
            acc = ((accs[0] + accs[1]) + (accs[2] + accs[3])) + (
                (accs[4] + accs[5]) + (accs[6] + accs[7])
            )
            logits = acc * (1.0 / TEMPERATURE)
            t = rbufs[ph][pl.ds(off, 16)]
            w_raw = wbufs[ph][pl.ds(off, 16)]
            n_vec = n0 + c * CH + off + lane
            w = jnp.where(n_vec < l_vec, w_raw, 0.0)
            bce = jnp.maximum(logits, 0.0) - logits * t + _softplus_neg_abs(jnp.abs(logits))

            y1 = bce * w - wl_c
            t1 = wl_a + y1
            wl_c_new = (t1 - wl_a) - y1
            y2 = w - w_c
            t2 = w_a + y2
            w_c_new = (t2 - w_a) - y2
            return t1, wl_c_new, t2, w_c_new

        return plsc.parallel_loop(
            0, GROUPS, carry=(wl_acc[0], wl_acc[1], w_acc[0], w_acc[1])
        )(group_body)

    def pair_body(i, carry):
        wl, wlc, w, wc = carry
        for ph in range(2):
            c = 2 * i + ph
            wait(c, ph)
            wl, wlc, w, wc = compute_chunk(c, ph, (wl, wlc), (w, wc))

            @pl.when(c + 2 < NCH)
            def _():
                start(c + 2, ph)

        return wl, wlc, w, wc

    zero = jnp.zeros((16,), jnp.float32)
    wl, _, w, _ = lax.fori_loop(0, NCH // 2, pair_body, (zero, zero, zero, zero))

    acc_buf[pl.ds(0, 16)] = wl
    acc_buf[pl.ds(16, 16)] = w
    pltpu.sync_copy(acc_buf, part_out.at[pl.ds(wid * 32, 32)])


def _tc_body(len_ref, o_ref, s_ref, w_ref, r_ref, wl_out, w_out, wl_acc, w_acc):
    b = pl.program_id(0)
    j = pl.program_id(1)

    o = o_ref[0]
    s = s_ref[0]
    logits = jnp.sum(o * s, axis=0) * (1.0 / TEMPERATURE)

    n_idx = j * TCCH + lax.broadcasted_iota(jnp.int32, (TCCH,), 0)
    valid = (n_idx < len_ref[b]).astype(jnp.float32)
    w = w_ref[b] * valid
    t = r_ref[b]
    bce = jnp.maximum(logits, 0.0) - logits * t + jnp.log1p(jnp.exp(-jnp.abs(logits)))

    @pl.when((b == 0) & (j == 0))
    def _init():
        wl_acc[...] = jnp.zeros_like(wl_acc)
        w_acc[...] = jnp.zeros_like(w_acc)

    wl_acc[...] += (bce * w).reshape(wl_acc.shape)
    w_acc[...] += w.reshape(w_acc.shape)

    @pl.when((b == B_TC - 1) & (j == (N // TCCH) - 1))
    def _final():
        wl_out[0, 0] = jnp.sum(wl_acc[...])
        w_out[0, 0] = jnp.sum(w_acc[...])


@jax.jit
def _run(lengths, o_t, s_t, w2, r2):
    mesh = plsc.VectorSubcoreMesh(core_axis_name="c", subcore_axis_name="s")
    sc = pl.kernel(
        _sc_body,
        out_type=jax.ShapeDtypeStruct((NW * 32,), jnp.float32),
        mesh=mesh,
        scratch_types=[
            pltpu.VMEM((D, CH), jnp.float32),
            pltpu.VMEM((D, CH), jnp.float32),
            pltpu.VMEM((D, CH), jnp.float32),
            pltpu.VMEM((D, CH), jnp.float32),
            pltpu.VMEM((CH,), jnp.float32),
            pltpu.VMEM((CH,), jnp.float32),
            pltpu.VMEM((CH,), jnp.float32),
            pltpu.VMEM((CH,), jnp.float32),
            pltpu.VMEM((16,), jnp.int32),
            pltpu.VMEM((32,), jnp.float32),
            pltpu.SemaphoreType.DMA,
            pltpu.SemaphoreType.DMA,
        ],
        compiler_params=pltpu.CompilerParams(needs_layout_passes=False),
    )
    sc_parts = sc(lengths, o_t, s_t, w2, r2).reshape(NW, 2, 16)

    wl_tc, w_tc = pl.pallas_call(
        _tc_body,
        grid=(B_TC, N // TCCH),
        in_specs=[
            pl.BlockSpec(memory_space=pltpu.SMEM),
            pl.BlockSpec((1, D, TCCH), lambda b, j: (b, 0, j)),
            pl.BlockSpec((1, D, TCCH), lambda b, j: (b, 0, j)),
            pl.BlockSpec((B, N), lambda b, j: (0, 0)),
            pl.BlockSpec((B, N), lambda b, j: (0, 0)),
        ],
        out_specs=[
            pl.BlockSpec(memory_space=pltpu.SMEM),
            pl.BlockSpec(memory_space=pltpu.SMEM),
        ],
        out_shape=[
            jax.ShapeDtypeStruct((1, 1), jnp.float32),
            jax.ShapeDtypeStruct((1, 1), jnp.float32),
        ],
        scratch_shapes=[
            pltpu.VMEM((8, TCCH // 8), jnp.float32),
            pltpu.VMEM((8, TCCH // 8), jnp.float32),
        ],
    )(lengths, o_t, s_t, w2, r2)

    num = jnp.sum(sc_parts[:, 0, :]) + wl_tc[0, 0]
    den = jnp.sum(sc_parts[:, 1, :]) + w_tc[0, 0]
    return num / den


def kernel(lengths, output_embeddings, supervision_ids, supervision_embeddings, supervision_weights, supervision_ratings):
    del supervision_ids
    o_t = output_embeddings.transpose(0, 2, 1)
    s_t = supervision_embeddings.transpose(0, 2, 1)
    return _run(lengths, o_t, s_t, supervision_weights, supervision_ratings)
